# baseline (speedup 1.0000x reference)
import jax
import jax.numpy as jnp
from jax.experimental import pallas as pl

_B, _C, _N = 8, 190, 16384
_NB = 512


def _conf_body(x_ref, out_ref):
    m1 = x_ref[0]
    m2 = jnp.full((_B, _NB), -jnp.inf, jnp.float32)
    for c in range(1, _C):
        v = x_ref[c]
        m2 = jnp.maximum(m2, jnp.minimum(m1, v))
        m1 = jnp.maximum(m1, v)
    conf = jnp.exp(1.0 - m1 / (m2 + 1e-8))
    out_ref[0] = conf


def kernel(sim_mat):
    xt = jnp.transpose(sim_mat, (1, 0, 2))
    nblk = _N // _NB
    conf = pl.pallas_call(
        _conf_body,
        grid=(nblk,),
        in_specs=[pl.BlockSpec((_C, _B, _NB), lambda n: (0, 0, n))],
        out_specs=pl.BlockSpec((1, _B, _NB), lambda n: (n, 0, 0)),
        out_shape=jax.ShapeDtypeStruct((nblk, _B, _NB), jnp.float32),
    )(xt)
    return jnp.mean(conf, axis=(0, 2))


# speedup vs baseline: 1.2765x; 1.2765x over previous
import jax
import jax.numpy as jnp
from jax.experimental import pallas as pl

_B, _C, _N = 8, 190, 16384
_NB = 1024
_LANES = 128


def _conf_body(x_ref, out_ref):
    m1 = x_ref[0]
    m2 = jnp.full((_B, _NB), -jnp.inf, jnp.float32)
    for c in range(1, _C):
        v = x_ref[c]
        m2 = jnp.maximum(m2, jnp.minimum(m1, v))
        m1 = jnp.maximum(m1, v)
    conf = jnp.exp(1.0 - m1 / (m2 + 1e-8))
    psum = jnp.zeros((_B, _LANES), jnp.float32)
    for k in range(_NB // _LANES):
        psum = psum + conf[:, k * _LANES:(k + 1) * _LANES]

    @pl.when(pl.program_id(0) == 0)
    def _init():
        out_ref[...] = jnp.zeros((_B, _LANES), jnp.float32)

    out_ref[...] += psum


def kernel(sim_mat):
    xt = jnp.transpose(sim_mat, (1, 0, 2))
    nblk = _N // _NB
    psums = pl.pallas_call(
        _conf_body,
        grid=(nblk,),
        in_specs=[pl.BlockSpec((_C, _B, _NB), lambda n: (0, 0, n))],
        out_specs=pl.BlockSpec((_B, _LANES), lambda n: (0, 0)),
        out_shape=jax.ShapeDtypeStruct((_B, _LANES), jnp.float32),
    )(xt)
    return psums.sum(axis=-1) / _N
